# Initial kernel scaffold; baseline (speedup 1.0000x reference)
import jax
import jax.numpy as jnp
from jax.experimental import pallas as pl


def kernel(X, edge_index_feat, edge_index_spatial, Wl1, Wr1, att1, b1, Wl2, Wr2, att2, b2, M, alpha_p):
    raise NotImplementedError("write your pallas kernel here")


# speedup vs baseline: 8.2043x; 8.2043x over previous
import functools

import jax
import jax.numpy as jnp
from jax import lax
from jax.experimental import pallas as pl
from jax.experimental.pallas import tpu as pltpu
from jax.experimental.pallas import tpu_sc as plsc

_NC = 2
_NS = 16
_NW = _NC * _NS
_L = 16


def _zero_vmem_2d(ref, rows, cols):
    z = jnp.zeros((_L,), jnp.float32)

    def body(r, c):
        for cb in range(cols // _L):
            ref[r, pl.ds(cb * _L, _L)] = z
        return c

    lax.fori_loop(0, rows, body, 0)


def _make_edge_pass(n_nodes, n_edges, dl, dr, je, chunk, with_compute=True):
    ept = n_edges // _NW
    nch = ept // chunk
    assert ept * _NW == n_edges and nch * chunk == ept and chunk % _L == 0
    ntc = 10
    rpt = n_nodes // ntc
    zrows = 40
    assert rpt * ntc == n_nodes and rpt % zrows == 0 and rpt % 8 == 0

    mesh = plsc.VectorSubcoreMesh(core_axis_name="c", subcore_axis_name="s")

    scratch = [
        pltpu.VMEM((nch, chunk), jnp.int32),
        pltpu.VMEM((nch, chunk), jnp.int32),
        pltpu.VMEM((chunk, dl), jnp.float32),
        pltpu.VMEM((chunk, dr), jnp.float32),
        pltpu.VMEM((chunk, dl), jnp.float32),
        pltpu.VMEM((max(je, 1), _L), jnp.float32),
        pltpu.VMEM((zrows, dl), jnp.float32),
        pltpu.VMEM_SHARED((n_nodes, dl), jnp.float32),
        pltpu.SemaphoreType.DMA,
        pltpu.SemaphoreType.DMA,
    ]

    @functools.partial(
        pl.kernel,
        out_type=jax.ShapeDtypeStruct((_NC, n_nodes, dl), jnp.float32),
        mesh=mesh,
        scratch_types=scratch,
        compiler_params=pltpu.CompilerParams(
            needs_layout_passes=False, use_tc_tiling_on_sc=False),
    )
    def kern(xl_hbm, xr_hbm, src_hbm, dst_hbm, att_hbm, out_hbm,
             srcv, dstv, xlbuf, xrbuf, outbuf, attbuf, zbuf, acc,
             sem1, sem2):
        cid = lax.axis_index("c")
        sid = lax.axis_index("s")
        wid = sid * _NC + cid

        pltpu.sync_copy(src_hbm.at[wid], srcv)
        pltpu.sync_copy(dst_hbm.at[wid], dstv)
        if with_compute:
            pltpu.sync_copy(att_hbm, attbuf)

        _zero_vmem_2d(outbuf, chunk, dl)
        _zero_vmem_2d(zbuf, zrows, dl)

        rbase = sid * rpt
        @pl.when(sid < ntc)
        def _():
            def zacc(i, c):
                pltpu.sync_copy(zbuf, acc.at[pl.ds(rbase + i * zrows, zrows)])
                return c
            lax.fori_loop(0, rpt // zrows, zacc, 0)
        plsc.subcore_barrier()

        def chunk_body(ci, c):
            cp1 = pltpu.async_copy(xl_hbm.at[srcv.at[ci]], xlbuf, sem1)
            if with_compute:
                cp2 = pltpu.async_copy(xr_hbm.at[dstv.at[ci]], xrbuf, sem2)
                cp2.wait()
            cp1.wait()

            if with_compute:
                def grp(g, gc):
                    offs = g * _L + lax.iota(jnp.int32, _L)
                    e = jnp.zeros((_L,), jnp.float32)
                    for j in range(je):
                        jv = jnp.full((_L,), j, jnp.int32)
                        vl = plsc.load_gather(xlbuf, [offs, jv])
                        vr = plsc.load_gather(xrbuf, [offs, jv])
                        t = vl + vr
                        t = jnp.where(t >= 0.0, t, t * 0.2)
                        e = e + attbuf[j] * t
                    p = jnp.exp(e)
                    for j in range(je + 1):
                        jv = jnp.full((_L,), j, jnp.int32)
                        vl = plsc.load_gather(xlbuf, [offs, jv])
                        plsc.store_scatter(outbuf, [offs, jv], p * vl)
                    return gc
                lax.fori_loop(0, chunk // _L, grp, 0)
                pltpu.sync_copy(outbuf, acc.at[dstv.at[ci]], add=True)
            else:
                pltpu.sync_copy(xlbuf, acc.at[dstv.at[ci]], add=True)
            return c

        lax.fori_loop(0, nch, chunk_body, 0)
        plsc.subcore_barrier()

        @pl.when(sid < ntc)
        def _():
            pltpu.sync_copy(acc.at[pl.ds(rbase, rpt)],
                            out_hbm.at[cid, pl.ds(rbase, rpt)])

    return kern


def _mm_kernel(x_ref, w_ref, o_ref):
    o_ref[...] = jnp.dot(x_ref[...], w_ref[...],
                         preferred_element_type=jnp.float32)


def _layer1_epilogue_kernel(parts_ref, xl_ref, xr_ref, att_ref, b_ref,
                            w_ref, y_ref):
    xl = xl_ref[...]
    xr = xr_ref[...]
    t = xl + xr
    t = jnp.where(t >= 0.0, t, t * 0.2)
    e = jnp.sum(t * att_ref[...], axis=1, keepdims=True)
    p = jnp.exp(e)
    ps = parts_ref[0] + parts_ref[1]
    hid = xl.shape[1]
    num = ps[:, :hid] + p * xl
    den = ps[:, hid:hid + 1] + p + 1e-16
    h = num / den + b_ref[...]
    h = jnp.where(h > 0.0, h, jnp.exp(h) - 1.0)
    y_ref[...] = jnp.dot(h, w_ref[...], preferred_element_type=jnp.float32)


def _layer2_epilogue_kernel(parts_ref, yl_ref, yr_ref, att_ref, b_ref,
                            z_ref):
    yl = yl_ref[...]
    yr = yr_ref[...]
    t = yl + yr
    t = jnp.where(t >= 0.0, t, t * 0.2)
    e = jnp.sum(t * att_ref[...], axis=1, keepdims=True)
    p = jnp.exp(e)
    ps = parts_ref[0] + parts_ref[1]
    k = yl.shape[1]
    num = ps[:, :k] + p * yl
    den = ps[:, k:k + 1] + p + 1e-16
    logits = num / den + b_ref[...]
    m = jnp.max(logits, axis=1, keepdims=True)
    z = jnp.exp(logits - m)
    z_ref[...] = z / jnp.sum(z, axis=1, keepdims=True)


def _blur_kernel(z_ref, zparts_ref, m_ref, alpha_ref, o_ref):
    k = z_ref.shape[1]
    zs = zparts_ref[0][:, :k] + zparts_ref[1][:, :k]
    s = z_ref[...] + alpha_ref[0, 0] * zs
    mr = jnp.maximum(m_ref[...], 0.0)
    o_ref[...] = jnp.dot(s, mr, preferred_element_type=jnp.float32)


def kernel(X, edge_index_feat, edge_index_spatial, Wl1, Wr1, att1, b1,
           Wl2, Wr2, att2, b2, M, alpha_p):
    n, f_in = X.shape
    hid = Wl1.shape[1]
    k = Wl2.shape[1]
    e_num = edge_index_feat.shape[1]

    d1 = 80
    d2 = 32
    chunk = 80
    rb = 2000

    nch = (e_num // _NW) // chunk
    src_f = edge_index_feat[0].reshape(_NW, nch, chunk)
    dst_f = edge_index_feat[1].reshape(_NW, nch, chunk)
    row_s = edge_index_spatial[0].reshape(_NW, nch, chunk)
    col_s = edge_index_spatial[1].reshape(_NW, nch, chunk)

    wcat1 = jnp.concatenate([Wl1, Wr1], axis=1)
    xw = pl.pallas_call(
        _mm_kernel,
        grid=(n // rb,),
        in_specs=[pl.BlockSpec((rb, f_in), lambda i: (i, 0)),
                  pl.BlockSpec((f_in, 2 * hid), lambda i: (0, 0))],
        out_specs=pl.BlockSpec((rb, 2 * hid), lambda i: (i, 0)),
        out_shape=jax.ShapeDtypeStruct((n, 2 * hid), jnp.float32),
    )(X, wcat1)
    xl1 = xw[:, :hid]
    xr1 = xw[:, hid:]

    ones_col = jnp.ones((n, 1), jnp.float32)
    xl1_pad = jnp.concatenate(
        [xl1, ones_col, jnp.zeros((n, d1 - hid - 1), jnp.float32)], axis=1)
    att1_sp = jnp.broadcast_to(att1[:, None], (hid, _L))

    pass1 = _make_edge_pass(n, e_num, d1, hid, hid, chunk)
    parts1 = pass1(xl1_pad, xr1, src_f, dst_f, att1_sp)

    wcat2 = jnp.concatenate([Wl2, Wr2], axis=1)
    y2 = pl.pallas_call(
        _layer1_epilogue_kernel,
        grid=(n // rb,),
        in_specs=[pl.BlockSpec((_NC, rb, d1), lambda i: (0, i, 0)),
                  pl.BlockSpec((rb, hid), lambda i: (i, 0)),
                  pl.BlockSpec((rb, hid), lambda i: (i, 0)),
                  pl.BlockSpec((1, hid), lambda i: (0, 0)),
                  pl.BlockSpec((1, hid), lambda i: (0, 0)),
                  pl.BlockSpec((hid, 2 * k), lambda i: (0, 0))],
        out_specs=pl.BlockSpec((rb, 2 * k), lambda i: (i, 0)),
        out_shape=jax.ShapeDtypeStruct((n, 2 * k), jnp.float32),
    )(parts1, xl1, xr1, att1[None, :], b1[None, :], wcat2)
    yl2 = y2[:, :k]
    yr2 = y2[:, k:]

    yl2_pad = jnp.concatenate(
        [yl2, ones_col, jnp.zeros((n, d2 - k - 1), jnp.float32)], axis=1)
    yr2_pad = jnp.concatenate(
        [yr2, jnp.zeros((n, d2 - k), jnp.float32)], axis=1)
    att2_sp = jnp.broadcast_to(att2[:, None], (k, _L))

    pass2 = _make_edge_pass(n, e_num, d2, d2, k, chunk)
    parts2 = pass2(yl2_pad, yr2_pad, src_f, dst_f, att2_sp)

    Z = pl.pallas_call(
        _layer2_epilogue_kernel,
        grid=(n // rb,),
        in_specs=[pl.BlockSpec((_NC, rb, d2), lambda i: (0, i, 0)),
                  pl.BlockSpec((rb, k), lambda i: (i, 0)),
                  pl.BlockSpec((rb, k), lambda i: (i, 0)),
                  pl.BlockSpec((1, k), lambda i: (0, 0)),
                  pl.BlockSpec((1, k), lambda i: (0, 0))],
        out_specs=pl.BlockSpec((rb, k), lambda i: (i, 0)),
        out_shape=jax.ShapeDtypeStruct((n, k), jnp.float32),
    )(parts2, yl2, yr2, att2[None, :], b2[None, :])

    z_pad = jnp.concatenate([Z, jnp.zeros((n, d2 - k), jnp.float32)], axis=1)
    pass3 = _make_edge_pass(n, e_num, d2, d2, 1, chunk, with_compute=False)
    zparts = pass3(z_pad, z_pad, col_s, row_s,
                   jnp.zeros((1, _L), jnp.float32))

    x_hat = pl.pallas_call(
        _blur_kernel,
        grid=(n // rb,),
        in_specs=[pl.BlockSpec((rb, k), lambda i: (i, 0)),
                  pl.BlockSpec((_NC, rb, d2), lambda i: (0, i, 0)),
                  pl.BlockSpec((k, f_in), lambda i: (0, 0)),
                  pl.BlockSpec(memory_space=pltpu.SMEM)],
        out_specs=pl.BlockSpec((rb, f_in), lambda i: (i, 0)),
        out_shape=jax.ShapeDtypeStruct((n, f_in), jnp.float32),
    )(Z, zparts, M, jnp.reshape(alpha_p, (1, 1)))

    return (Z, x_hat)


# speedup vs baseline: 30.8293x; 3.7577x over previous
import functools

import jax
import jax.numpy as jnp
from jax import lax
from jax.experimental import pallas as pl
from jax.experimental.pallas import tpu as pltpu
from jax.experimental.pallas import tpu_sc as plsc

_NC = 2
_NS = 16
_NW = _NC * _NS
_L = 16


def _zero_vmem_2d(ref, rows, cols):
    z = jnp.zeros((_L,), jnp.float32)

    def body(r, c):
        for cb in range(cols // _L):
            ref[r, pl.ds(cb * _L, _L)] = z
        return c

    lax.fori_loop(0, rows, body, 0)


def _make_edge_pass(n_nodes, n_edges, dl, dr, nf, chunk, unroll=4,
                    with_compute=True):
    ept = n_edges // _NW
    nch = ept // chunk
    assert ept * _NW == n_edges and nch * chunk == ept and nch % 2 == 0
    nvl = dl // _L
    assert nvl * _L == dl and nf * _L <= dr + _L - 1 and nf <= nvl
    ntc = 10
    rpt = n_nodes // ntc
    zrows = 8
    assert rpt * ntc == n_nodes and rpt % zrows == 0 and rpt % 8 == 0

    mesh = plsc.VectorSubcoreMesh(core_axis_name="c", subcore_axis_name="s")

    scratch = [
        pltpu.VMEM((nch, chunk), jnp.int32),
        pltpu.VMEM((nch, chunk), jnp.int32),
        pltpu.VMEM((chunk, dl), jnp.float32),
        pltpu.VMEM((chunk, dl), jnp.float32),
        pltpu.VMEM((chunk, dr), jnp.float32),
        pltpu.VMEM((chunk, dr), jnp.float32),
        pltpu.VMEM((chunk, dl), jnp.float32),
        pltpu.VMEM((chunk, dl), jnp.float32),
        pltpu.VMEM((max(nf, 1) * _L,), jnp.float32),
        pltpu.VMEM((zrows, dl), jnp.float32),
        pltpu.VMEM_SHARED((n_nodes, dl), jnp.float32),
        pltpu.SemaphoreType.DMA,
        pltpu.SemaphoreType.DMA,
        pltpu.SemaphoreType.DMA,
        pltpu.SemaphoreType.DMA,
        pltpu.SemaphoreType.DMA,
        pltpu.SemaphoreType.DMA,
    ]

    @functools.partial(
        pl.kernel,
        out_type=jax.ShapeDtypeStruct((_NC, n_nodes, dl), jnp.float32),
        mesh=mesh,
        scratch_types=scratch,
        compiler_params=pltpu.CompilerParams(
            needs_layout_passes=False, use_tc_tiling_on_sc=False),
    )
    def kern(xl_hbm, xr_hbm, src_hbm, dst_hbm, att_hbm, out_hbm,
             srcv, dstv, xl0, xl1, xr0, xr1, ob0, ob1, attbuf, zbuf, acc,
             sl0, sl1, sr0, sr1, ss0, ss1):
        cid = lax.axis_index("c")
        sid = lax.axis_index("s")
        wid = sid * _NC + cid
        xlb = (xl0, xl1)
        xrb = (xr0, xr1)
        obb = (ob0, ob1)
        slb = (sl0, sl1)
        srb = (sr0, sr1)
        ssb = (ss0, ss1)

        pltpu.sync_copy(src_hbm.at[wid], srcv)
        pltpu.sync_copy(dst_hbm.at[wid], dstv)
        if with_compute:
            pltpu.sync_copy(att_hbm, attbuf)
            attv = [attbuf[pl.ds(k * _L, _L)] for k in range(nf)]

        _zero_vmem_2d(zbuf, zrows, dl)

        rbase = sid * rpt
        @pl.when(sid < ntc)
        def _():
            def zacc(i, c):
                pltpu.sync_copy(zbuf, acc.at[pl.ds(rbase + i * zrows, zrows)])
                return c
            lax.fori_loop(0, rpt // zrows, zacc, 0)
        plsc.subcore_barrier()

        def issue_gathers(ci, b):
            pltpu.async_copy(xl_hbm.at[srcv.at[ci]], xlb[b], slb[b])
            if with_compute:
                pltpu.async_copy(xr_hbm.at[dstv.at[ci]], xrb[b], srb[b])

        def wait_gathers(ci, b):
            pltpu.make_async_copy(xl_hbm.at[srcv.at[ci]], xlb[b],
                                  slb[b]).wait()
            if with_compute:
                pltpu.make_async_copy(xr_hbm.at[dstv.at[ci]], xrb[b],
                                      srb[b]).wait()

        def wait_scatter(ci, b):
            src = obb[b] if with_compute else xlb[b]
            pltpu.make_async_copy(src, acc.at[dstv.at[ci]], ssb[b]).wait()

        def compute(b):
            xlr = xlb[b]
            xrr = xrb[b]
            obr = obb[b]

            @plsc.parallel_loop(0, chunk, 1, unroll=unroll)
            def _(e):
                vls = [xlr[e, pl.ds(k * _L, _L)] for k in range(nvl)]
                ea = None
                for k in range(nf):
                    t = vls[k] + xrr[e, pl.ds(k * _L, _L)]
                    t = jnp.where(t >= 0.0, t, t * 0.2)
                    t = t * attv[k]
                    ea = t if ea is None else ea + t
                p = jnp.exp(jnp.full((_L,), jnp.sum(ea), jnp.float32))
                for k in range(nvl):
                    obr[e, pl.ds(k * _L, _L)] = p * vls[k]

        issue_gathers(0, 0)

        if with_compute:
            def pair_body(i, c):
                a = 2 * i
                for b in (0, 1):
                    ci = a + b
                    if b == 0:
                        issue_gathers(ci + 1, 1)
                    wait_gathers(ci, b)
                    @pl.when(i > 0)
                    def _():
                        wait_scatter(ci, b)
                    compute(b)
                    pltpu.async_copy(obb[b], acc.at[dstv.at[ci]], ssb[b],
                                     add=True)
                    if b == 1:
                        @pl.when(ci + 1 < nch)
                        def _():
                            issue_gathers(ci + 1, 0)
                return c

            lax.fori_loop(0, nch // 2, pair_body, 0)
            wait_scatter(nch - 2, 0)
            wait_scatter(nch - 1, 1)
        else:
            def pair_body(i, c):
                a = 2 * i
                issue_gathers(a + 1, 1)
                wait_gathers(a, 0)
                pltpu.async_copy(xlb[0], acc.at[dstv.at[a]], ssb[0],
                                 add=True)
                wait_scatter(a, 0)
                @pl.when(a + 2 < nch)
                def _():
                    issue_gathers(a + 2, 0)
                wait_gathers(a + 1, 1)
                pltpu.async_copy(xlb[1], acc.at[dstv.at[a + 1]], ssb[1],
                                 add=True)
                wait_scatter(a + 1, 1)
                return c

            lax.fori_loop(0, nch // 2, pair_body, 0)
        plsc.subcore_barrier()

        @pl.when(sid < ntc)
        def _():
            pltpu.sync_copy(acc.at[pl.ds(rbase, rpt)],
                            out_hbm.at[cid, pl.ds(rbase, rpt)])

    return kern


def _mm_kernel(x_ref, w_ref, o_ref):
    o_ref[...] = jnp.dot(x_ref[...], w_ref[...],
                         preferred_element_type=jnp.float32)


def _layer1_epilogue_kernel(parts_ref, xl_ref, xr_ref, att_ref, b_ref,
                            w_ref, y_ref):
    xl = xl_ref[...]
    xr = xr_ref[...]
    t = xl + xr
    t = jnp.where(t >= 0.0, t, t * 0.2)
    e = jnp.sum(t * att_ref[...], axis=1, keepdims=True)
    p = jnp.exp(e)
    ps = parts_ref[0] + parts_ref[1]
    hid = xl.shape[1]
    num = ps[:, :hid] + p * xl
    den = ps[:, hid:hid + 1] + p + 1e-16
    h = num / den + b_ref[...]
    h = jnp.where(h > 0.0, h, jnp.exp(h) - 1.0)
    y_ref[...] = jnp.dot(h, w_ref[...], preferred_element_type=jnp.float32)


def _layer2_epilogue_kernel(parts_ref, yl_ref, yr_ref, att_ref, b_ref,
                            z_ref):
    yl = yl_ref[...]
    yr = yr_ref[...]
    t = yl + yr
    t = jnp.where(t >= 0.0, t, t * 0.2)
    e = jnp.sum(t * att_ref[...], axis=1, keepdims=True)
    p = jnp.exp(e)
    ps = parts_ref[0] + parts_ref[1]
    k = yl.shape[1]
    num = ps[:, :k] + p * yl
    den = ps[:, k:k + 1] + p + 1e-16
    logits = num / den + b_ref[...]
    m = jnp.max(logits, axis=1, keepdims=True)
    z = jnp.exp(logits - m)
    z_ref[...] = z / jnp.sum(z, axis=1, keepdims=True)


def _blur_kernel(z_ref, zparts_ref, m_ref, alpha_ref, o_ref):
    k = z_ref.shape[1]
    zs = zparts_ref[0][:, :k] + zparts_ref[1][:, :k]
    s = z_ref[...] + alpha_ref[0, 0] * zs
    mr = jnp.maximum(m_ref[...], 0.0)
    o_ref[...] = jnp.dot(s, mr, preferred_element_type=jnp.float32)


def kernel(X, edge_index_feat, edge_index_spatial, Wl1, Wr1, att1, b1,
           Wl2, Wr2, att2, b2, M, alpha_p):
    n, f_in = X.shape
    hid = Wl1.shape[1]
    k = Wl2.shape[1]
    e_num = edge_index_feat.shape[1]

    d1 = 80
    d2 = 32
    chunk = 100
    rb = 2000

    nch = (e_num // _NW) // chunk
    src_f = edge_index_feat[0].reshape(_NW, nch, chunk)
    dst_f = edge_index_feat[1].reshape(_NW, nch, chunk)
    row_s = edge_index_spatial[0].reshape(_NW, nch, chunk)
    col_s = edge_index_spatial[1].reshape(_NW, nch, chunk)

    wcat1 = jnp.concatenate([Wl1, Wr1], axis=1)
    xw = pl.pallas_call(
        _mm_kernel,
        grid=(n // rb,),
        in_specs=[pl.BlockSpec((rb, f_in), lambda i: (i, 0)),
                  pl.BlockSpec((f_in, 2 * hid), lambda i: (0, 0))],
        out_specs=pl.BlockSpec((rb, 2 * hid), lambda i: (i, 0)),
        out_shape=jax.ShapeDtypeStruct((n, 2 * hid), jnp.float32),
    )(X, wcat1)
    xl1 = xw[:, :hid]
    xr1 = xw[:, hid:]

    ones_col = jnp.ones((n, 1), jnp.float32)
    xl1_pad = jnp.concatenate(
        [xl1, ones_col, jnp.zeros((n, d1 - hid - 1), jnp.float32)], axis=1)

    pass1 = _make_edge_pass(n, e_num, d1, hid, hid // _L, chunk)
    parts1 = pass1(xl1_pad, xr1, src_f, dst_f, att1)

    wcat2 = jnp.concatenate([Wl2, Wr2], axis=1)
    y2 = pl.pallas_call(
        _layer1_epilogue_kernel,
        grid=(n // rb,),
        in_specs=[pl.BlockSpec((_NC, rb, d1), lambda i: (0, i, 0)),
                  pl.BlockSpec((rb, hid), lambda i: (i, 0)),
                  pl.BlockSpec((rb, hid), lambda i: (i, 0)),
                  pl.BlockSpec((1, hid), lambda i: (0, 0)),
                  pl.BlockSpec((1, hid), lambda i: (0, 0)),
                  pl.BlockSpec((hid, 2 * k), lambda i: (0, 0))],
        out_specs=pl.BlockSpec((rb, 2 * k), lambda i: (i, 0)),
        out_shape=jax.ShapeDtypeStruct((n, 2 * k), jnp.float32),
    )(parts1, xl1, xr1, att1[None, :], b1[None, :], wcat2)
    yl2 = y2[:, :k]
    yr2 = y2[:, k:]

    yl2_pad = jnp.concatenate(
        [yl2, ones_col, jnp.zeros((n, d2 - k - 1), jnp.float32)], axis=1)
    yr2_pad = jnp.concatenate(
        [yr2, jnp.zeros((n, d2 - k), jnp.float32)], axis=1)
    att2_pad = jnp.concatenate([att2, jnp.zeros((d2 - k,), jnp.float32)])

    pass2 = _make_edge_pass(n, e_num, d2, d2, d2 // _L, chunk)
    parts2 = pass2(yl2_pad, yr2_pad, src_f, dst_f, att2_pad)

    Z = pl.pallas_call(
        _layer2_epilogue_kernel,
        grid=(n // rb,),
        in_specs=[pl.BlockSpec((_NC, rb, d2), lambda i: (0, i, 0)),
                  pl.BlockSpec((rb, k), lambda i: (i, 0)),
                  pl.BlockSpec((rb, k), lambda i: (i, 0)),
                  pl.BlockSpec((1, k), lambda i: (0, 0)),
                  pl.BlockSpec((1, k), lambda i: (0, 0))],
        out_specs=pl.BlockSpec((rb, k), lambda i: (i, 0)),
        out_shape=jax.ShapeDtypeStruct((n, k), jnp.float32),
    )(parts2, yl2, yr2, att2[None, :], b2[None, :])

    z_pad = jnp.concatenate([Z, jnp.zeros((n, d2 - k), jnp.float32)], axis=1)
    pass3 = _make_edge_pass(n, e_num, d2, d2, 0, chunk, with_compute=False)
    zparts = pass3(z_pad, z_pad, col_s, row_s,
                   jnp.zeros((_L,), jnp.float32))

    x_hat = pl.pallas_call(
        _blur_kernel,
        grid=(n // rb,),
        in_specs=[pl.BlockSpec((rb, k), lambda i: (i, 0)),
                  pl.BlockSpec((_NC, rb, d2), lambda i: (0, i, 0)),
                  pl.BlockSpec((k, f_in), lambda i: (0, 0)),
                  pl.BlockSpec(memory_space=pltpu.SMEM)],
        out_specs=pl.BlockSpec((rb, f_in), lambda i: (i, 0)),
        out_shape=jax.ShapeDtypeStruct((n, f_in), jnp.float32),
    )(Z, zparts, M, jnp.reshape(alpha_p, (1, 1)))

    return (Z, x_hat)


# speedup vs baseline: 36.4235x; 1.1815x over previous
import functools

import jax
import jax.numpy as jnp
from jax import lax
from jax.experimental import pallas as pl
from jax.experimental.pallas import tpu as pltpu
from jax.experimental.pallas import tpu_sc as plsc

_NC = 2
_NS = 16
_NW = _NC * _NS
_L = 16


def _zero_vmem_2d(ref, rows, cols):
    z = jnp.zeros((_L,), jnp.float32)

    def body(r, c):
        for cb in range(cols // _L):
            ref[r, pl.ds(cb * _L, _L)] = z
        return c

    lax.fori_loop(0, rows, body, 0)


def _make_edge_pass(n_nodes, n_edges, dl, dr, nf, chunk, unroll=4,
                    with_compute=True, dacc=None):
    ept = n_edges // _NW
    nch = ept // chunk
    assert ept * _NW == n_edges and nch * chunk == ept and nch % 2 == 0
    nvl = dl // _L
    if dacc is None:
        dacc = dl
    flag_store = dacc > dl
    assert nvl * _L == dl and nf * _L <= dr + _L - 1 and nf <= nvl
    assert dacc in (dl, dl + _L)
    ntc = 10
    rpt = n_nodes // ntc
    zrows = 8
    assert rpt * ntc == n_nodes and rpt % zrows == 0 and rpt % 8 == 0

    mesh = plsc.VectorSubcoreMesh(core_axis_name="c", subcore_axis_name="s")

    scratch = [
        pltpu.VMEM((nch, chunk), jnp.int32),
        pltpu.VMEM((nch, chunk), jnp.int32),
        pltpu.VMEM((chunk, dl), jnp.float32),
        pltpu.VMEM((chunk, dl), jnp.float32),
        pltpu.VMEM((chunk, dr), jnp.float32),
        pltpu.VMEM((chunk, dr), jnp.float32),
        pltpu.VMEM((chunk, dacc), jnp.float32),
        pltpu.VMEM((chunk, dacc), jnp.float32),
        pltpu.VMEM((max(nf, 1) * _L,), jnp.float32),
        pltpu.VMEM((zrows, dacc), jnp.float32),
        pltpu.VMEM_SHARED((n_nodes, dacc), jnp.float32),
        pltpu.SemaphoreType.DMA,
        pltpu.SemaphoreType.DMA,
        pltpu.SemaphoreType.DMA,
        pltpu.SemaphoreType.DMA,
        pltpu.SemaphoreType.DMA,
        pltpu.SemaphoreType.DMA,
    ]

    @functools.partial(
        pl.kernel,
        out_type=jax.ShapeDtypeStruct((_NC, n_nodes, dacc), jnp.float32),
        mesh=mesh,
        scratch_types=scratch,
        compiler_params=pltpu.CompilerParams(
            needs_layout_passes=False, use_tc_tiling_on_sc=False),
    )
    def kern(xl_hbm, xr_hbm, src_hbm, dst_hbm, att_hbm, out_hbm,
             srcv, dstv, xl0, xl1, xr0, xr1, ob0, ob1, attbuf, zbuf, acc,
             sl0, sl1, sr0, sr1, ss0, ss1):
        cid = lax.axis_index("c")
        sid = lax.axis_index("s")
        wid = sid * _NC + cid
        xlb = (xl0, xl1)
        xrb = (xr0, xr1)
        obb = (ob0, ob1)
        slb = (sl0, sl1)
        srb = (sr0, sr1)
        ssb = (ss0, ss1)

        pltpu.sync_copy(src_hbm.at[wid], srcv)
        pltpu.sync_copy(dst_hbm.at[wid], dstv)
        if with_compute:
            pltpu.sync_copy(att_hbm, attbuf)
            attv = [attbuf[pl.ds(k * _L, _L)] for k in range(nf)]
        if flag_store:
            lane = lax.iota(jnp.int32, _L)
            flagv = jnp.where(lane == 0, 1.0, 0.0).astype(jnp.float32)

        _zero_vmem_2d(zbuf, zrows, dacc)

        rbase = sid * rpt
        @pl.when(sid < ntc)
        def _():
            def zacc(i, c):
                pltpu.sync_copy(zbuf, acc.at[pl.ds(rbase + i * zrows, zrows)])
                return c
            lax.fori_loop(0, rpt // zrows, zacc, 0)
        plsc.subcore_barrier()

        def issue_gathers(ci, b):
            pltpu.async_copy(xl_hbm.at[srcv.at[ci]], xlb[b], slb[b])
            if with_compute:
                pltpu.async_copy(xr_hbm.at[dstv.at[ci]], xrb[b], srb[b])

        def wait_gathers(ci, b):
            pltpu.make_async_copy(xl_hbm.at[srcv.at[ci]], xlb[b],
                                  slb[b]).wait()
            if with_compute:
                pltpu.make_async_copy(xr_hbm.at[dstv.at[ci]], xrb[b],
                                      srb[b]).wait()

        def wait_scatter(ci, b):
            src = obb[b] if with_compute else xlb[b]
            pltpu.make_async_copy(src, acc.at[dstv.at[ci]], ssb[b]).wait()

        def compute(b):
            xlr = xlb[b]
            xrr = xrb[b]
            obr = obb[b]

            @plsc.parallel_loop(0, chunk, 1, unroll=unroll)
            def _(e):
                vls = [xlr[e, pl.ds(k * _L, _L)] for k in range(nvl)]
                ea = None
                for k in range(nf):
                    t = vls[k] + xrr[e, pl.ds(k * _L, _L)]
                    t = jnp.where(t >= 0.0, t, t * 0.2)
                    t = t * attv[k]
                    ea = t if ea is None else ea + t
                p = jnp.exp(jnp.full((_L,), jnp.sum(ea), jnp.float32))
                for k in range(nvl):
                    obr[e, pl.ds(k * _L, _L)] = p * vls[k]
                if flag_store:
                    obr[e, pl.ds(nvl * _L, _L)] = p * flagv

        issue_gathers(0, 0)

        if with_compute:
            def pair_body(i, c):
                a = 2 * i
                for b in (0, 1):
                    ci = a + b
                    if b == 0:
                        issue_gathers(ci + 1, 1)
                    wait_gathers(ci, b)
                    @pl.when(i > 0)
                    def _():
                        wait_scatter(ci, b)
                    compute(b)
                    pltpu.async_copy(obb[b], acc.at[dstv.at[ci]], ssb[b],
                                     add=True)
                    if b == 1:
                        @pl.when(ci + 1 < nch)
                        def _():
                            issue_gathers(ci + 1, 0)
                return c

            lax.fori_loop(0, nch // 2, pair_body, 0)
            wait_scatter(nch - 2, 0)
            wait_scatter(nch - 1, 1)
        else:
            def pair_body(i, c):
                a = 2 * i
                issue_gathers(a + 1, 1)
                wait_gathers(a, 0)
                pltpu.async_copy(xlb[0], acc.at[dstv.at[a]], ssb[0],
                                 add=True)
                wait_scatter(a, 0)
                @pl.when(a + 2 < nch)
                def _():
                    issue_gathers(a + 2, 0)
                wait_gathers(a + 1, 1)
                pltpu.async_copy(xlb[1], acc.at[dstv.at[a + 1]], ssb[1],
                                 add=True)
                wait_scatter(a + 1, 1)
                return c

            lax.fori_loop(0, nch // 2, pair_body, 0)
        plsc.subcore_barrier()

        @pl.when(sid < ntc)
        def _():
            pltpu.sync_copy(acc.at[pl.ds(rbase, rpt)],
                            out_hbm.at[cid, pl.ds(rbase, rpt)])

    return kern


def _mm_kernel(x_ref, w_ref, o_ref):
    o_ref[...] = jnp.dot(x_ref[...], w_ref[...],
                         preferred_element_type=jnp.float32)


def _layer1_epilogue_kernel(parts_ref, xl_ref, xr_ref, att_ref, b_ref,
                            w_ref, y_ref):
    xl = xl_ref[...]
    xr = xr_ref[...]
    t = xl + xr
    t = jnp.where(t >= 0.0, t, t * 0.2)
    e = jnp.sum(t * att_ref[...], axis=1, keepdims=True)
    p = jnp.exp(e)
    ps = parts_ref[0] + parts_ref[1]
    hid = xl.shape[1]
    num = ps[:, :hid] + p * xl
    den = ps[:, hid:hid + 1] + p + 1e-16
    h = num / den + b_ref[...]
    h = jnp.where(h > 0.0, h, jnp.exp(h) - 1.0)
    y_ref[...] = jnp.dot(h, w_ref[...], preferred_element_type=jnp.float32)


def _layer2_epilogue_kernel(parts_ref, yl_ref, yr_ref, att_ref, b_ref,
                            z_ref):
    yl = yl_ref[...]
    yr = yr_ref[...]
    t = yl + yr
    t = jnp.where(t >= 0.0, t, t * 0.2)
    e = jnp.sum(t * att_ref[...], axis=1, keepdims=True)
    p = jnp.exp(e)
    ps = parts_ref[0] + parts_ref[1]
    k = yl.shape[1]
    num = ps[:, :k] + p * yl
    den = ps[:, k:k + 1] + p + 1e-16
    logits = num / den + b_ref[...]
    m = jnp.max(logits, axis=1, keepdims=True)
    z = jnp.exp(logits - m)
    z_ref[...] = z / jnp.sum(z, axis=1, keepdims=True)


def _blur_kernel(z_ref, zparts_ref, m_ref, alpha_ref, o_ref):
    k = z_ref.shape[1]
    zs = zparts_ref[0][:, :k] + zparts_ref[1][:, :k]
    s = z_ref[...] + alpha_ref[0, 0] * zs
    mr = jnp.maximum(m_ref[...], 0.0)
    o_ref[...] = jnp.dot(s, mr, preferred_element_type=jnp.float32)


def kernel(X, edge_index_feat, edge_index_spatial, Wl1, Wr1, att1, b1,
           Wl2, Wr2, att2, b2, M, alpha_p):
    n, f_in = X.shape
    hid = Wl1.shape[1]
    k = Wl2.shape[1]
    e_num = edge_index_feat.shape[1]

    d1 = 80
    d2 = 32
    chunk1 = 125
    chunk2 = 250
    rb = 2000

    nch1 = (e_num // _NW) // chunk1
    nch2 = (e_num // _NW) // chunk2
    src_f = edge_index_feat[0].reshape(_NW, nch1, chunk1)
    dst_f = edge_index_feat[1].reshape(_NW, nch1, chunk1)
    src_f2 = edge_index_feat[0].reshape(_NW, nch2, chunk2)
    dst_f2 = edge_index_feat[1].reshape(_NW, nch2, chunk2)
    row_s = edge_index_spatial[0].reshape(_NW, nch2, chunk2)
    col_s = edge_index_spatial[1].reshape(_NW, nch2, chunk2)

    wcat1 = jnp.concatenate([Wl1, Wr1], axis=1)
    xw = pl.pallas_call(
        _mm_kernel,
        grid=(n // rb,),
        in_specs=[pl.BlockSpec((rb, f_in), lambda i: (i, 0)),
                  pl.BlockSpec((f_in, 2 * hid), lambda i: (0, 0))],
        out_specs=pl.BlockSpec((rb, 2 * hid), lambda i: (i, 0)),
        out_shape=jax.ShapeDtypeStruct((n, 2 * hid), jnp.float32),
    )(X, wcat1)
    xl1 = xw[:, :hid]
    xr1 = xw[:, hid:]

    ones_col = jnp.ones((n, 1), jnp.float32)

    pass1 = _make_edge_pass(n, e_num, hid, hid, hid // _L, chunk1, dacc=d1)
    parts1 = pass1(xl1, xr1, src_f, dst_f, att1)

    wcat2 = jnp.concatenate([Wl2, Wr2], axis=1)
    y2 = pl.pallas_call(
        _layer1_epilogue_kernel,
        grid=(n // rb,),
        in_specs=[pl.BlockSpec((_NC, rb, d1), lambda i: (0, i, 0)),
                  pl.BlockSpec((rb, hid), lambda i: (i, 0)),
                  pl.BlockSpec((rb, hid), lambda i: (i, 0)),
                  pl.BlockSpec((1, hid), lambda i: (0, 0)),
                  pl.BlockSpec((1, hid), lambda i: (0, 0)),
                  pl.BlockSpec((hid, 2 * k), lambda i: (0, 0))],
        out_specs=pl.BlockSpec((rb, 2 * k), lambda i: (i, 0)),
        out_shape=jax.ShapeDtypeStruct((n, 2 * k), jnp.float32),
    )(parts1, xl1, xr1, att1[None, :], b1[None, :], wcat2)
    yl2 = y2[:, :k]
    yr2 = y2[:, k:]

    yl2_pad = jnp.concatenate(
        [yl2, ones_col, jnp.zeros((n, d2 - k - 1), jnp.float32)], axis=1)
    yr2_pad = jnp.concatenate(
        [yr2, jnp.zeros((n, d2 - k), jnp.float32)], axis=1)
    att2_pad = jnp.concatenate([att2, jnp.zeros((d2 - k,), jnp.float32)])

    pass2 = _make_edge_pass(n, e_num, d2, d2, d2 // _L, chunk2)
    parts2 = pass2(yl2_pad, yr2_pad, src_f2, dst_f2, att2_pad)

    Z = pl.pallas_call(
        _layer2_epilogue_kernel,
        grid=(n // rb,),
        in_specs=[pl.BlockSpec((_NC, rb, d2), lambda i: (0, i, 0)),
                  pl.BlockSpec((rb, k), lambda i: (i, 0)),
                  pl.BlockSpec((rb, k), lambda i: (i, 0)),
                  pl.BlockSpec((1, k), lambda i: (0, 0)),
                  pl.BlockSpec((1, k), lambda i: (0, 0))],
        out_specs=pl.BlockSpec((rb, k), lambda i: (i, 0)),
        out_shape=jax.ShapeDtypeStruct((n, k), jnp.float32),
    )(parts2, yl2, yr2, att2[None, :], b2[None, :])

    z_pad = jnp.concatenate([Z, jnp.zeros((n, d2 - k), jnp.float32)], axis=1)
    pass3 = _make_edge_pass(n, e_num, d2, d2, 0, chunk2, with_compute=False)
    zparts = pass3(z_pad, z_pad, col_s, row_s,
                   jnp.zeros((_L,), jnp.float32))

    x_hat = pl.pallas_call(
        _blur_kernel,
        grid=(n // rb,),
        in_specs=[pl.BlockSpec((rb, k), lambda i: (i, 0)),
                  pl.BlockSpec((_NC, rb, d2), lambda i: (0, i, 0)),
                  pl.BlockSpec((k, f_in), lambda i: (0, 0)),
                  pl.BlockSpec(memory_space=pltpu.SMEM)],
        out_specs=pl.BlockSpec((rb, f_in), lambda i: (i, 0)),
        out_shape=jax.ShapeDtypeStruct((n, f_in), jnp.float32),
    )(Z, zparts, M, jnp.reshape(alpha_p, (1, 1)))

    return (Z, x_hat)


# speedup vs baseline: 38.0928x; 1.0458x over previous
import functools

import jax
import jax.numpy as jnp
from jax import lax
from jax.experimental import pallas as pl
from jax.experimental.pallas import tpu as pltpu
from jax.experimental.pallas import tpu_sc as plsc

_NC = 2
_NS = 16
_NW = _NC * _NS
_L = 16


def _zero_vmem_2d(ref, rows, cols):
    z = jnp.zeros((_L,), jnp.float32)

    def body(r, c):
        for cb in range(cols // _L):
            ref[r, pl.ds(cb * _L, _L)] = z
        return c

    lax.fori_loop(0, rows, body, 0)


def _make_edge_pass(n_nodes, n_edges, dl, dr, nf, chunk, unroll=4,
                    with_compute=True, dacc=None):
    ept = n_edges // _NW
    nch = ept // chunk
    assert ept * _NW == n_edges and nch * chunk == ept and nch % 2 == 0
    nvl = dl // _L
    if dacc is None:
        dacc = dl
    flag_store = dacc > dl
    assert nvl * _L == dl and nf * _L <= dr + _L - 1 and nf <= nvl
    assert dacc in (dl, dl + _L)
    ntc = 10
    rpt = n_nodes // ntc
    zrows = 8
    assert rpt * ntc == n_nodes and rpt % zrows == 0 and rpt % 8 == 0

    mesh = plsc.VectorSubcoreMesh(core_axis_name="c", subcore_axis_name="s")

    scratch = [
        pltpu.VMEM((nch, chunk), jnp.int32),
        pltpu.VMEM((nch, chunk), jnp.int32),
        pltpu.VMEM((chunk, dl), jnp.float32),
        pltpu.VMEM((chunk, dl), jnp.float32),
        pltpu.VMEM((chunk, dr), jnp.float32),
        pltpu.VMEM((chunk, dr), jnp.float32),
        pltpu.VMEM((chunk, dacc), jnp.float32),
        pltpu.VMEM((chunk, dacc), jnp.float32),
        pltpu.VMEM((max(nf, 1) * _L,), jnp.float32),
        pltpu.VMEM((zrows, dacc), jnp.float32),
        pltpu.VMEM_SHARED((n_nodes, dacc), jnp.float32),
        pltpu.SemaphoreType.DMA,
        pltpu.SemaphoreType.DMA,
        pltpu.SemaphoreType.DMA,
        pltpu.SemaphoreType.DMA,
        pltpu.SemaphoreType.DMA,
        pltpu.SemaphoreType.DMA,
    ]

    @functools.partial(
        pl.kernel,
        out_type=jax.ShapeDtypeStruct((_NC, n_nodes, dacc), jnp.float32),
        mesh=mesh,
        scratch_types=scratch,
        compiler_params=pltpu.CompilerParams(
            needs_layout_passes=False, use_tc_tiling_on_sc=False),
    )
    def kern(xl_hbm, xr_hbm, src_hbm, dst_hbm, att_hbm, out_hbm,
             srcv, dstv, xl0, xl1, xr0, xr1, ob0, ob1, attbuf, zbuf, acc,
             sl0, sl1, sr0, sr1, ss0, ss1):
        cid = lax.axis_index("c")
        sid = lax.axis_index("s")
        wid = sid * _NC + cid
        xlb = (xl0, xl1)
        xrb = (xr0, xr1)
        obb = (ob0, ob1)
        slb = (sl0, sl1)
        srb = (sr0, sr1)
        ssb = (ss0, ss1)

        pltpu.sync_copy(src_hbm.at[wid], srcv)
        pltpu.sync_copy(dst_hbm.at[wid], dstv)
        if with_compute:
            pltpu.sync_copy(att_hbm, attbuf)
            attv = [attbuf[pl.ds(k * _L, _L)] for k in range(nf)]
        if flag_store:
            lane = lax.iota(jnp.int32, _L)
            flagv = jnp.where(lane == 0, 1.0, 0.0).astype(jnp.float32)

        _zero_vmem_2d(zbuf, zrows, dacc)

        rbase = sid * rpt
        @pl.when(sid < ntc)
        def _():
            def zacc(i, c):
                pltpu.sync_copy(zbuf, acc.at[pl.ds(rbase + i * zrows, zrows)])
                return c
            lax.fori_loop(0, rpt // zrows, zacc, 0)
        plsc.subcore_barrier()

        def issue_gathers(ci, b):
            pltpu.async_copy(xl_hbm.at[srcv.at[ci]], xlb[b], slb[b])
            if with_compute:
                pltpu.async_copy(xr_hbm.at[dstv.at[ci]], xrb[b], srb[b])

        def wait_gathers(ci, b):
            pltpu.make_async_copy(xl_hbm.at[srcv.at[ci]], xlb[b],
                                  slb[b]).wait()
            if with_compute:
                pltpu.make_async_copy(xr_hbm.at[dstv.at[ci]], xrb[b],
                                      srb[b]).wait()

        def wait_scatter(ci, b):
            src = obb[b] if with_compute else xlb[b]
            pltpu.make_async_copy(src, acc.at[dstv.at[ci]], ssb[b]).wait()

        def compute(b):
            xlr = xlb[b]
            xrr = xrb[b]
            obr = obb[b]

            @plsc.parallel_loop(0, chunk, 1, unroll=unroll)
            def _(e):
                vls = [xlr[e, pl.ds(k * _L, _L)] for k in range(nvl)]
                ea = None
                for k in range(nf):
                    t = vls[k] + xrr[e, pl.ds(k * _L, _L)]
                    t = jnp.where(t >= 0.0, t, t * 0.2)
                    t = t * attv[k]
                    ea = t if ea is None else ea + t
                p = jnp.exp(jnp.full((_L,), jnp.sum(ea), jnp.float32))
                for k in range(nvl):
                    obr[e, pl.ds(k * _L, _L)] = p * vls[k]
                if flag_store:
                    obr[e, pl.ds(nvl * _L, _L)] = p * flagv

        issue_gathers(0, 0)

        if with_compute:
            def pair_body(i, c):
                a = 2 * i
                for b in (0, 1):
                    ci = a + b
                    if b == 0:
                        issue_gathers(ci + 1, 1)
                    wait_gathers(ci, b)
                    @pl.when(i > 0)
                    def _():
                        wait_scatter(ci, b)
                    compute(b)
                    pltpu.async_copy(obb[b], acc.at[dstv.at[ci]], ssb[b],
                                     add=True)
                    if b == 1:
                        @pl.when(ci + 1 < nch)
                        def _():
                            issue_gathers(ci + 1, 0)
                return c

            lax.fori_loop(0, nch // 2, pair_body, 0)
            wait_scatter(nch - 2, 0)
            wait_scatter(nch - 1, 1)
        else:
            def pair_body(i, c):
                a = 2 * i
                issue_gathers(a + 1, 1)
                wait_gathers(a, 0)
                pltpu.async_copy(xlb[0], acc.at[dstv.at[a]], ssb[0],
                                 add=True)
                wait_scatter(a, 0)
                @pl.when(a + 2 < nch)
                def _():
                    issue_gathers(a + 2, 0)
                wait_gathers(a + 1, 1)
                pltpu.async_copy(xlb[1], acc.at[dstv.at[a + 1]], ssb[1],
                                 add=True)
                wait_scatter(a + 1, 1)
                return c

            lax.fori_loop(0, nch // 2, pair_body, 0)
        plsc.subcore_barrier()

        @pl.when(sid < ntc)
        def _():
            pltpu.sync_copy(acc.at[pl.ds(rbase, rpt)],
                            out_hbm.at[cid, pl.ds(rbase, rpt)])

    return kern


def _mm_split_kernel(x_ref, w_ref, xl_ref, xr_ref):
    hid = xl_ref.shape[1]
    xw = jnp.dot(x_ref[...], w_ref[...], preferred_element_type=jnp.float32)
    xl_ref[...] = xw[:, :hid]
    xr_ref[...] = xw[:, hid:]


def _layer1_epilogue_kernel(parts_ref, xl_ref, xr_ref, att_ref, b_ref,
                            w_ref, ylp_ref, yrp_ref):
    xl = xl_ref[...]
    xr = xr_ref[...]
    t = xl + xr
    t = jnp.where(t >= 0.0, t, t * 0.2)
    e = jnp.sum(t * att_ref[...], axis=1, keepdims=True)
    p = jnp.exp(e)
    ps = parts_ref[0] + parts_ref[1]
    hid = xl.shape[1]
    num = ps[:, :hid] + p * xl
    den = ps[:, hid:hid + 1] + p + 1e-16
    h = num / den + b_ref[...]
    h = jnp.where(h > 0.0, h, jnp.exp(h) - 1.0)
    y = jnp.dot(h, w_ref[...], preferred_element_type=jnp.float32)
    k = y.shape[1] // 2
    rows = y.shape[0]
    ones = jnp.ones((rows, 1), jnp.float32)
    zero = jnp.zeros((rows, 1), jnp.float32)
    ylp_ref[...] = jnp.concatenate([y[:, :k], ones, zero], axis=1)
    yrp_ref[...] = jnp.concatenate([y[:, k:], zero, zero], axis=1)


def _layer2_epilogue_kernel(parts_ref, ylp_ref, yrp_ref, att_ref, b_ref,
                            z_ref, zp_ref):
    ylp = ylp_ref[...]
    yrp = yrp_ref[...]
    t = ylp + yrp
    t = jnp.where(t >= 0.0, t, t * 0.2)
    e = jnp.sum(t * att_ref[...], axis=1, keepdims=True)
    p = jnp.exp(e)
    ps = parts_ref[0] + parts_ref[1]
    k = z_ref.shape[1]
    num = ps[:, :k] + p * ylp[:, :k]
    den = ps[:, k:k + 1] + p + 1e-16
    logits = num / den + b_ref[...]
    m = jnp.max(logits, axis=1, keepdims=True)
    z = jnp.exp(logits - m)
    z = z / jnp.sum(z, axis=1, keepdims=True)
    z_ref[...] = z
    zp_ref[...] = jnp.concatenate(
        [z, jnp.zeros((z.shape[0], zp_ref.shape[1] - k), jnp.float32)],
        axis=1)


def _blur_kernel(z_ref, zparts_ref, m_ref, alpha_ref, o_ref):
    k = z_ref.shape[1]
    zs = zparts_ref[0][:, :k] + zparts_ref[1][:, :k]
    s = z_ref[...] + alpha_ref[0, 0] * zs
    mr = jnp.maximum(m_ref[...], 0.0)
    o_ref[...] = jnp.dot(s, mr, preferred_element_type=jnp.float32)


def kernel(X, edge_index_feat, edge_index_spatial, Wl1, Wr1, att1, b1,
           Wl2, Wr2, att2, b2, M, alpha_p):
    n, f_in = X.shape
    hid = Wl1.shape[1]
    k = Wl2.shape[1]
    e_num = edge_index_feat.shape[1]

    d1 = 80
    d2 = 32
    chunk1 = 125
    chunk2 = 250
    rb = 2000

    nch1 = (e_num // _NW) // chunk1
    nch2 = (e_num // _NW) // chunk2
    src_f = edge_index_feat[0].reshape(_NW, nch1, chunk1)
    dst_f = edge_index_feat[1].reshape(_NW, nch1, chunk1)
    src_f2 = src_f.reshape(_NW, nch2, chunk2)
    dst_f2 = dst_f.reshape(_NW, nch2, chunk2)
    row_s = edge_index_spatial[0].reshape(_NW, nch2, chunk2)
    col_s = edge_index_spatial[1].reshape(_NW, nch2, chunk2)

    wcat1 = jnp.concatenate([Wl1, Wr1], axis=1)
    xl1, xr1 = pl.pallas_call(
        _mm_split_kernel,
        grid=(n // rb,),
        in_specs=[pl.BlockSpec((rb, f_in), lambda i: (i, 0)),
                  pl.BlockSpec((f_in, 2 * hid), lambda i: (0, 0))],
        out_specs=[pl.BlockSpec((rb, hid), lambda i: (i, 0)),
                   pl.BlockSpec((rb, hid), lambda i: (i, 0))],
        out_shape=[jax.ShapeDtypeStruct((n, hid), jnp.float32),
                   jax.ShapeDtypeStruct((n, hid), jnp.float32)],
    )(X, wcat1)

    pass1 = _make_edge_pass(n, e_num, hid, hid, hid // _L, chunk1, dacc=d1)
    parts1 = pass1(xl1, xr1, src_f, dst_f, att1)

    wcat2 = jnp.concatenate([Wl2, Wr2], axis=1)
    yl2_pad, yr2_pad = pl.pallas_call(
        _layer1_epilogue_kernel,
        grid=(n // rb,),
        in_specs=[pl.BlockSpec((_NC, rb, d1), lambda i: (0, i, 0)),
                  pl.BlockSpec((rb, hid), lambda i: (i, 0)),
                  pl.BlockSpec((rb, hid), lambda i: (i, 0)),
                  pl.BlockSpec((1, hid), lambda i: (0, 0)),
                  pl.BlockSpec((1, hid), lambda i: (0, 0)),
                  pl.BlockSpec((hid, 2 * k), lambda i: (0, 0))],
        out_specs=[pl.BlockSpec((rb, d2), lambda i: (i, 0)),
                   pl.BlockSpec((rb, d2), lambda i: (i, 0))],
        out_shape=[jax.ShapeDtypeStruct((n, d2), jnp.float32),
                   jax.ShapeDtypeStruct((n, d2), jnp.float32)],
    )(parts1, xl1, xr1, att1[None, :], b1[None, :], wcat2)

    att2_pad = jnp.concatenate([att2, jnp.zeros((d2 - k,), jnp.float32)])

    pass2 = _make_edge_pass(n, e_num, d2, d2, d2 // _L, chunk2)
    parts2 = pass2(yl2_pad, yr2_pad, src_f2, dst_f2, att2_pad)

    Z, z_pad = pl.pallas_call(
        _layer2_epilogue_kernel,
        grid=(n // rb,),
        in_specs=[pl.BlockSpec((_NC, rb, d2), lambda i: (0, i, 0)),
                  pl.BlockSpec((rb, d2), lambda i: (i, 0)),
                  pl.BlockSpec((rb, d2), lambda i: (i, 0)),
                  pl.BlockSpec((1, d2), lambda i: (0, 0)),
                  pl.BlockSpec((1, k), lambda i: (0, 0))],
        out_specs=[pl.BlockSpec((rb, k), lambda i: (i, 0)),
                   pl.BlockSpec((rb, d2), lambda i: (i, 0))],
        out_shape=[jax.ShapeDtypeStruct((n, k), jnp.float32),
                   jax.ShapeDtypeStruct((n, d2), jnp.float32)],
    )(parts2, yl2_pad, yr2_pad, att2_pad[None, :], b2[None, :])

    pass3 = _make_edge_pass(n, e_num, d2, d2, 0, chunk2, with_compute=False)
    zparts = pass3(z_pad, z_pad, col_s, row_s,
                   jnp.zeros((_L,), jnp.float32))

    x_hat = pl.pallas_call(
        _blur_kernel,
        grid=(n // rb,),
        in_specs=[pl.BlockSpec((rb, k), lambda i: (i, 0)),
                  pl.BlockSpec((_NC, rb, d2), lambda i: (0, i, 0)),
                  pl.BlockSpec((k, f_in), lambda i: (0, 0)),
                  pl.BlockSpec(memory_space=pltpu.SMEM)],
        out_specs=pl.BlockSpec((rb, f_in), lambda i: (i, 0)),
        out_shape=jax.ShapeDtypeStruct((n, f_in), jnp.float32),
    )(Z, zparts, M, jnp.reshape(alpha_p, (1, 1)))

    return (Z, x_hat)
